# baseline (speedup 1.0000x reference)
import functools

import jax
import jax.numpy as jnp
from jax import lax
from jax.experimental import pallas as pl
from jax.experimental.pallas import tpu as pltpu
from jax.experimental.pallas import tpu_sc as plsc

N = 10000
E = 160000
D = 256
S = 1024
NC = 2
NS = 16
NW = NC * NS
EPT = E // NW
HIST = 10240
SEL_CAP = 5248
DUMP = S
BIN_CAP = 5504
OCAP = 6784
SPT = S // NS

_sc_mesh = functools.partial(
    plsc.VectorSubcoreMesh, core_axis_name="c", subcore_axis_name="s",
    num_cores=NC, num_subcores=NS)


def _i16():
  return lax.iota(jnp.int32, 16)


def _dyngather(x, idx):
  return lax.gather(
      x, idx[:, None],
      dimension_numbers=lax.GatherDimensionNumbers(
          offset_dims=(), collapsed_slice_dims=(0,), start_index_map=(0,)),
      slice_sizes=(1,), mode=lax.GatherScatterMode.PROMISE_IN_BOUNDS)


def _sc_deg(edge_dst):
  @functools.partial(
      pl.kernel,
      out_type=jax.ShapeDtypeStruct((NC, HIST), jnp.float32),
      mesh=_sc_mesh(),
      compiler_params=pltpu.CompilerParams(needs_layout_passes=False),
      scratch_types=[
          pltpu.VMEM((5008,), jnp.int32),
          pltpu.VMEM((HIST,), jnp.float32),
          pltpu.VMEM((640,), jnp.float32),
          pltpu.VMEM((640,), jnp.float32),
          pltpu.VMEM_SHARED((NS, HIST), jnp.float32),
      ],
  )
  def k(dst_hbm, out_hbm, dstb, hist, accs, tmp, banks):
    c = lax.axis_index("c")
    s = lax.axis_index("s")
    wid = c * NS + s

    pltpu.sync_copy(dst_hbm.at[pl.ds(wid * EPT, EPT)], dstb.at[pl.ds(0, EPT)])

    def zh(i, _):
      hist[pl.ds(i * 16, 16)] = jnp.zeros((16,), jnp.float32)
      return 0
    lax.fori_loop(0, HIST // 16, zh, 0)

    def body(i, _):
      base = i * 16
      valid = (base + _i16()) < EPT
      v = jnp.where(valid, dstb[pl.ds(base, 16)], 0)
      ks, _vs, om = plsc.sort_key_val(v, _i16(), mask=valid)
      ksan = jnp.where(om, ks, -1 - _i16())
      prev = _dyngather(ksan, jnp.maximum(_i16() - 1, 0))
      first = (_i16() == 0) | (ksan != prev)
      pos_first = jnp.where(first, _i16(), 16)
      sm = -lax.rev(plsc.cummax(lax.rev(-pos_first, (0,))), (0,))
      nxt = _dyngather(sm, jnp.minimum(_i16() + 1, 15))
      nxt = jnp.where(_i16() == 15, 16, nxt)
      runlen = (nxt - pos_first).astype(jnp.float32)
      plsc.addupdate_scatter(hist, [ks], runlen, mask=first & om)
      return 0
    lax.fori_loop(0, 313, body, 0)

    pltpu.sync_copy(hist, banks.at[s])
    plsc.subcore_barrier()

    pltpu.sync_copy(banks.at[0, pl.ds(s * 640, 640)], accs)
    def mb(b, _):
      pltpu.sync_copy(banks.at[b, pl.ds(s * 640, 640)], tmp)
      def av(i, _2):
        plsc.addupdate(accs.at[pl.ds(i * 16, 16)], tmp[pl.ds(i * 16, 16)])
        return 0
      lax.fori_loop(0, 40, av, 0)
      return 0
    lax.fori_loop(1, NS, mb, 0)
    pltpu.sync_copy(accs, out_hbm.at[c, pl.ds(s * 640, 640)])

  return k(edge_dst)


def _tc_y(x, Wg, deg_col):
  nb = pl.cdiv(N, 128)

  def body(x_ref, d_ref, wg_ref, y_ref, di_ref):
    di = lax.rsqrt(d_ref[...])
    y_ref[...] = jnp.dot(x_ref[...] * di, wg_ref[...],
                         preferred_element_type=jnp.float32)
    di_ref[...] = di

  return pl.pallas_call(
      body,
      grid=(nb,),
      in_specs=[
          pl.BlockSpec((128, D), lambda i: (i, 0)),
          pl.BlockSpec((128, 1), lambda i: (i, 0)),
          pl.BlockSpec((D, D), lambda i: (0, 0)),
      ],
      out_specs=[
          pl.BlockSpec((128, D), lambda i: (i, 0)),
          pl.BlockSpec((128, 1), lambda i: (i, 0)),
      ],
      out_shape=[
          jax.ShapeDtypeStruct((N, D), jnp.float32),
          jax.ShapeDtypeStruct((N, 1), jnp.float32),
      ],
  )(x, deg_col, Wg)


def _sc_agg(y, edge_src, edge_dst, node_index):
  out_type = (
      jax.ShapeDtypeStruct((S, D), jnp.float32),
      jax.ShapeDtypeStruct((S, D), jnp.float32),
      jax.ShapeDtypeStruct((S,), jnp.int32),
  )

  @functools.partial(
      pl.kernel,
      out_type=out_type,
      mesh=_sc_mesh(),
      compiler_params=pltpu.CompilerParams(needs_layout_passes=False),
      scratch_types=[
          pltpu.VMEM((N,), jnp.int32),
          pltpu.VMEM((S,), jnp.int32),
          pltpu.VMEM((5008,), jnp.int32),
          pltpu.VMEM((5008,), jnp.int32),
          pltpu.VMEM((SEL_CAP,), jnp.int32),
          pltpu.VMEM((SEL_CAP,), jnp.int32),
          pltpu.VMEM((BIN_CAP,), jnp.int32),
          pltpu.VMEM((BIN_CAP,), jnp.int32),
          pltpu.VMEM((OCAP,), jnp.int32),
          pltpu.VMEM((OCAP,), jnp.int32),
          pltpu.VMEM((32,), jnp.int32),
          pltpu.VMEM((128,), jnp.int32),
          pltpu.VMEM((128, D), jnp.float32),
          pltpu.VMEM((SPT + 1, D), jnp.float32),
          pltpu.VMEM((S,), jnp.int32),
          pltpu.VMEM((32,), jnp.int32),
          pltpu.VMEM((NS * 32,), jnp.int32),
          pltpu.VMEM_SHARED((NS * BIN_CAP,), jnp.int32),
          pltpu.VMEM_SHARED((NS * BIN_CAP,), jnp.int32),
          pltpu.VMEM_SHARED((NS * 32,), jnp.int32),
          pltpu.SemaphoreType.DMA,
      ],
  )
  def k(y_hbm, src_hbm, dst_hbm, ni_hbm, acc0_hbm, acc1_hbm, csel_hbm,
        slot, nbuf, srcb, dstb, selsrc, selslot, bsrc, bslot, osrc, oslot,
        bc, idxg, rows, acc, csb, cbuf, cntv, psrc, pslot, cbank, sem):
    c = lax.axis_index("c")
    s = lax.axis_index("s")
    wid = c * NS + s

    pltpu.sync_copy(ni_hbm, nbuf)
    pltpu.sync_copy(src_hbm.at[pl.ds(wid * EPT, EPT)],
                    srcb.at[pl.ds(0, EPT)])
    pltpu.sync_copy(dst_hbm.at[pl.ds(wid * EPT, EPT)],
                    dstb.at[pl.ds(0, EPT)])

    def zr(r, _):
      def zv(v, _2):
        acc[r, pl.ds(v * 16, 16)] = jnp.zeros((16,), jnp.float32)
        return 0
      lax.fori_loop(0, D // 16, zv, 0)
      return 0
    lax.fori_loop(0, SPT + 1, zr, 0)

    def si(i, _):
      slot[pl.ds(i * 16, 16)] = jnp.full((16,), -1, jnp.int32)
      return 0
    lax.fori_loop(0, N // 16, si, 0)
    def sw(i, _):
      nv = nbuf[pl.ds(i * 16, 16)]
      jv = jnp.full((16,), i * 16, jnp.int32) + _i16()
      plsc.store_scatter(slot, [nv], jv)
      return 0
    lax.fori_loop(0, S // 16, sw, 0)

    j0 = wid * (S // NW)
    for k2 in range(2):
      sv = nbuf[pl.ds(j0 + k2 * 16, 16)]
      selsrc[pl.ds(k2 * 16, 16)] = sv
      selslot[pl.ds(k2 * 16, 16)] = (
          jnp.full((16,), j0 + k2 * 16, jnp.int32) + _i16())

    def flt(i, cnt):
      base = i * 16
      valid = (base + _i16()) < EPT
      dstv = jnp.where(valid, dstb[pl.ds(base, 16)], 0)
      srcv = srcb[pl.ds(base, 16)]
      slotv = plsc.load_gather(slot, [dstv])
      m = valid & (slotv >= 0)
      plsc.store_compressed(selsrc.at[pl.ds(cnt, 16)], srcv, mask=m)
      plsc.store_compressed(selslot.at[pl.ds(cnt, 16)], slotv, mask=m)
      return cnt + jnp.max(plsc.all_reduce_population_count(m))
    cnt = lax.fori_loop(0, 313, flt, jnp.int32(32))

    cnt_al = (cnt // 16) * 16
    keep = _i16() < (cnt - cnt_al)
    v = selsrc[pl.ds(cnt_al, 16)]
    selsrc[pl.ds(cnt_al, 16)] = jnp.where(keep, v, 0)
    v = selslot[pl.ds(cnt_al, 16)]
    selslot[pl.ds(cnt_al, 16)] = jnp.where(keep, v, DUMP)
    def pad(k2, _):
      off = cnt_al + 16 + k2 * 16
      selsrc[pl.ds(off, 16)] = jnp.zeros((16,), jnp.int32)
      selslot[pl.ds(off, 16)] = jnp.full((16,), DUMP, jnp.int32)
      return 0
    lax.fori_loop(0, 8, pad, 0)
    cntp = ((cnt + 127) // 128) * 128

    bc[pl.ds(0, 16)] = jnp.zeros((16,), jnp.int32)
    bc[pl.ds(16, 16)] = jnp.zeros((16,), jnp.int32)
    trips = cntp // 16
    def oh(t, _):
      ow = selslot[pl.ds(t * 16, 16)] >> 6
      ks, _vs = plsc.sort_key_val(ow, _i16())
      prev = _dyngather(ks, jnp.maximum(_i16() - 1, 0))
      first = (_i16() == 0) | (ks != prev)
      pos_first = jnp.where(first, _i16(), 16)
      sm = -lax.rev(plsc.cummax(lax.rev(-pos_first, (0,))), (0,))
      nxt = jnp.where(_i16() == 15, 16,
                      _dyngather(sm, jnp.minimum(_i16() + 1, 15)))
      plsc.addupdate_scatter(bc, [ks], nxt - pos_first, mask=first)
      return 0
    lax.fori_loop(0, trips, oh, 0)

    binc = bc[pl.ds(0, 16)]
    binc16 = ((binc + 15) >> 4) << 4
    incl = plsc.cumsum(binc16)
    excl = jnp.where(_i16() == 0, 0,
                     _dyngather(incl, jnp.maximum(_i16() - 1, 0)))
    total = jnp.max(incl)

    cbuf[pl.ds(0, 16)] = excl
    cbuf[pl.ds(16, 16)] = binc16
    pltpu.sync_copy(cbuf, cbank.at[pl.ds(s * 32, 32)])

    def sc2(t, cur):
      slotv = selslot[pl.ds(t * 16, 16)]
      srcv = selsrc[pl.ds(t * 16, 16)]
      ow = slotv >> 6
      newcur = []
      for o in range(16):
        m = ow == o
        plsc.store_compressed(bsrc.at[pl.ds(cur[o], 16)], srcv, mask=m)
        plsc.store_compressed(bslot.at[pl.ds(cur[o], 16)], slotv & 63, mask=m)
        newcur.append(cur[o] + jnp.max(plsc.all_reduce_population_count(m)))
      return tuple(newcur)
    curf = lax.fori_loop(0, trips, sc2,
                         tuple(excl[o] for o in range(16)))

    for o in range(16):
      @pl.when(curf[o] < excl[o] + binc16[o])
      def _(o=o):
        al = (curf[o] // 16) * 16
        keep3 = _i16() < (curf[o] - al)
        q = bsrc[pl.ds(al, 16)]
        bsrc[pl.ds(al, 16)] = jnp.where(keep3, q, 0)
        q = bslot[pl.ds(al, 16)]
        bslot[pl.ds(al, 16)] = jnp.where(keep3, q, SPT)

    def pub(g, _):
      pltpu.sync_copy(bsrc.at[pl.ds(g * 128, 128)],
                      psrc.at[pl.ds(s * BIN_CAP + g * 128, 128)])
      pltpu.sync_copy(bslot.at[pl.ds(g * 128, 128)],
                      pslot.at[pl.ds(s * BIN_CAP + g * 128, 128)])
      return 0
    lax.fori_loop(0, (total + 127) // 128, pub, 0)
    plsc.subcore_barrier()
    pltpu.sync_copy(cbank, cntv)

    def flush(fp):
      fp = pl.multiple_of(fp, 16)
      def cp(k2, _):
        idxg[pl.ds(k2 * 16, 16)] = osrc[pl.ds(fp + k2 * 16, 16)]
        return 0
      lax.fori_loop(0, 8, cp, 0)
      pltpu.async_copy(y_hbm.at[idxg], rows, sem).wait()
      def pe(eb, _):
        sls = oslot[pl.ds(fp + eb * 16, 16)]
        for l in range(16):
          sle = sls[l]
          for vv in range(D // 16):
            plsc.addupdate(acc.at[sle, pl.ds(vv * 16, 16)],
                           rows[eb * 16 + l, pl.ds(vv * 16, 16)])
        return 0
      lax.fori_loop(0, 8, pe, 0)
      return 0

    def bank(b, carry):
      fp0, co0 = carry
      offv = cntv[pl.ds(b * 32, 16)]
      lenv = cntv[pl.ds(b * 32 + 16, 16)]
      sv16 = jnp.full((16,), s, jnp.int32)
      my_off = pl.multiple_of(jnp.max(_dyngather(offv, sv16)), 16)
      my_len = pl.multiple_of(jnp.max(_dyngather(lenv, sv16)), 16)
      co0a = pl.multiple_of(co0, 16)
      def cpy(g, _):
        po = pl.multiple_of(b * BIN_CAP + my_off + g * 128, 16)
        oo = pl.multiple_of(co0a + g * 128, 16)
        pltpu.sync_copy(psrc.at[pl.ds(po, 128)], osrc.at[pl.ds(oo, 128)])
        pltpu.sync_copy(pslot.at[pl.ds(po, 128)], oslot.at[pl.ds(oo, 128)])
        return 0
      lax.fori_loop(0, (my_len + 127) // 128, cpy, 0)
      co1 = co0 + my_len
      nfl = (co1 - fp0) // 128
      def fl(g, _):
        flush(fp0 + g * 128)
        return 0
      lax.fori_loop(0, nfl, fl, 0)
      fp1 = fp0 + nfl * 128
      def compact(op2):
        fp4, co4 = op2
        def cpc(k2, _):
          q = osrc[pl.ds(fp4 + k2 * 16, 16)]
          osrc[pl.ds(k2 * 16, 16)] = q
          q = oslot[pl.ds(fp4 + k2 * 16, 16)]
          oslot[pl.ds(k2 * 16, 16)] = q
          return 0
        lax.fori_loop(0, 9, cpc, 0)
        return (jnp.int32(0), co4 - fp4)
      return lax.cond(fp1 >= 1024, compact, lambda op2: op2, (fp1, co1))
    fp, co = lax.fori_loop(0, NS, bank, (jnp.int32(0), jnp.int32(0)))

    co_al = (co // 16) * 16
    keep2 = _i16() < (co - co_al)
    v2 = osrc[pl.ds(co_al, 16)]
    osrc[pl.ds(co_al, 16)] = jnp.where(keep2, v2, 0)
    v2 = oslot[pl.ds(co_al, 16)]
    oslot[pl.ds(co_al, 16)] = jnp.where(keep2, v2, SPT)
    def pad2(k2, _):
      off = co_al + 16 + k2 * 16
      osrc[pl.ds(off, 16)] = jnp.zeros((16,), jnp.int32)
      oslot[pl.ds(off, 16)] = jnp.full((16,), SPT, jnp.int32)
      return 0
    lax.fori_loop(0, 8, pad2, 0)
    cop = ((co + 127) // 128) * 128
    def dr(g, _):
      flush(fp + g * 128)
      return 0
    lax.fori_loop(0, (cop - fp) // 128, dr, 0)

    @pl.when(c == 0)
    def _():
      pltpu.sync_copy(acc.at[pl.ds(0, SPT)], acc0_hbm.at[pl.ds(s * SPT, SPT)])

    @pl.when(c == 1)
    def _():
      pltpu.sync_copy(acc.at[pl.ds(0, SPT)], acc1_hbm.at[pl.ds(s * SPT, SPT)])

    @pl.when(wid == 0)
    def _():
      def cw(i, _):
        nv = nbuf[pl.ds(i * 16, 16)]
        csb[pl.ds(i * 16, 16)] = plsc.load_gather(slot, [nv])
        return 0
      lax.fori_loop(0, S // 16, cw, 0)
      pltpu.sync_copy(csb, csel_hbm)

  return k(y, edge_src, edge_dst, node_index)


def _sc_sites(acc0, acc1, csel, node_index, dinv, bg):
  SPW = S // NW

  @functools.partial(
      pl.kernel,
      out_type=jax.ShapeDtypeStruct((S, D), jnp.float32),
      mesh=_sc_mesh(),
      compiler_params=pltpu.CompilerParams(needs_layout_passes=False),
      scratch_types=[
          pltpu.VMEM((SPW,), jnp.int32),
          pltpu.VMEM((SPW,), jnp.int32),
          pltpu.VMEM((N,), jnp.float32),
          pltpu.VMEM((SPW, D), jnp.float32),
          pltpu.VMEM((SPW, D), jnp.float32),
          pltpu.VMEM((SPW, D), jnp.float32),
          pltpu.VMEM((D,), jnp.float32),
          pltpu.SemaphoreType.DMA,
      ],
  )
  def k(a0_hbm, a1_hbm, cs_hbm, ni_hbm, di_hbm, bg_hbm, out_hbm,
        cb, nb, db, ra, rb, ob, bgb, sem):
    c = lax.axis_index("c")
    s = lax.axis_index("s")
    wid = c * NS + s
    j0 = wid * SPW

    pltpu.sync_copy(cs_hbm.at[pl.ds(j0, SPW)], cb)
    pltpu.sync_copy(ni_hbm.at[pl.ds(j0, SPW)], nb)
    pltpu.sync_copy(bg_hbm, bgb)
    pltpu.sync_copy(di_hbm, db)
    pltpu.async_copy(a0_hbm.at[cb], ra, sem).wait()
    pltpu.async_copy(a1_hbm.at[cb], rb, sem).wait()

    def site(i, _):
      nv = nb[pl.ds((i // 16) * 16, 16)]
      dv = plsc.load_gather(db, [nv])
      dsp = _dyngather(dv, jnp.full((16,), i % 16, jnp.int32))
      def vec(v, _2):
        a = ra[i, pl.ds(v * 16, 16)]
        b = rb[i, pl.ds(v * 16, 16)]
        g = bgb[pl.ds(v * 16, 16)]
        val = (a + b) * dsp + g
        ob[i, pl.ds(v * 16, 16)] = jnp.maximum(val, val * 0.01)
        return 0
      lax.fori_loop(0, D // 16, vec, 0)
      return 0
    lax.fori_loop(0, SPW, site, 0)

    pltpu.sync_copy(ob, out_hbm.at[pl.ds(j0, SPW)])

  return k(acc0, acc1, csel, node_index, dinv, bg)


def _tc_mlp(hs, W1, b1, W2, b2, W3, b3, Wo_pad, bo_pad):
  def body(h_ref, w1, b1r, w2, b2r, w3, b3r, wo, bor, o_ref):
    h = h_ref[...]
    h = jnp.dot(h, w1[...], preferred_element_type=jnp.float32) + b1r[...]
    h = jnp.maximum(h, h * 0.01)
    h = jnp.dot(h, w2[...], preferred_element_type=jnp.float32) + b2r[...]
    h = jnp.maximum(h, h * 0.01)
    h = jnp.dot(h, w3[...], preferred_element_type=jnp.float32) + b3r[...]
    h = jnp.maximum(h, h * 0.01)
    z = jnp.dot(h, wo[...], preferred_element_type=jnp.float32) + bor[...]
    col = lax.broadcasted_iota(jnp.int32, (128, 128), 1)
    z = jnp.where(col < 10, z, -1e30)
    z = z - jnp.max(z, axis=1, keepdims=True)
    e = jnp.exp(z)
    o_ref[...] = e / jnp.sum(e, axis=1, keepdims=True)

  full = lambda shape: pl.BlockSpec(shape, lambda i: tuple(0 for _ in shape))
  return pl.pallas_call(
      body,
      grid=(S // 128,),
      in_specs=[
          pl.BlockSpec((128, D), lambda i: (i, 0)),
          full((D, 128)), full((1, 128)),
          full((128, 128)), full((1, 128)),
          full((128, 64)), full((1, 64)),
          full((64, 128)), full((1, 128)),
      ],
      out_specs=pl.BlockSpec((128, 128), lambda i: (i, 0)),
      out_shape=jax.ShapeDtypeStruct((S, 128), jnp.float32),
  )(hs, W1, b1.reshape(1, -1), W2, b2.reshape(1, -1),
    W3, b3.reshape(1, -1), Wo_pad, bo_pad)


def kernel(x, edge_index, node_index, Wg, bg, W1, b1, W2, b2, W3, b3, Wo, bo):
  edge_src = edge_index[0]
  edge_dst = edge_index[1]
  deg_parts = _sc_deg(edge_dst)
  deg_col = (deg_parts[0, :N] + deg_parts[1, :N] + 1.0).reshape(N, 1)
  y, dinv_col = _tc_y(x, Wg, deg_col)
  acc0, acc1, csel = _sc_agg(y, edge_src, edge_dst, node_index)
  hs = _sc_sites(acc0, acc1, csel, node_index, dinv_col.reshape(N), bg)
  Wo_pad = jnp.pad(Wo, ((0, 0), (0, 128 - Wo.shape[1])))
  bo_pad = jnp.pad(bo, (0, 128 - bo.shape[0])).reshape(1, -1)
  out = _tc_mlp(hs, W1, b1, W2, b2, W3, b3, Wo_pad, bo_pad)
  return out[:, :bo.shape[0]]


# speedup vs baseline: 1.3888x; 1.3888x over previous
import functools

import jax
import jax.numpy as jnp
from jax import lax
from jax.experimental import pallas as pl
from jax.experimental.pallas import tpu as pltpu
from jax.experimental.pallas import tpu_sc as plsc

N = 10000
E = 160000
D = 256
S = 1024
NC = 2
NS = 16
NW = NC * NS
EPT = E // NW
HIST = 10240
SEL_CAP = 6144
DUMP = S
BIN_CAP = 5504
OCAP = 1328
SPT = S // NS

_sc_mesh = functools.partial(
    plsc.VectorSubcoreMesh, core_axis_name="c", subcore_axis_name="s",
    num_cores=NC, num_subcores=NS)


def _i16():
  return lax.iota(jnp.int32, 16)


def _dyngather(x, idx):
  return lax.gather(
      x, idx[:, None],
      dimension_numbers=lax.GatherDimensionNumbers(
          offset_dims=(), collapsed_slice_dims=(0,), start_index_map=(0,)),
      slice_sizes=(1,), mode=lax.GatherScatterMode.PROMISE_IN_BOUNDS)


def _sc_deg(edge_dst):
  @functools.partial(
      pl.kernel,
      out_type=jax.ShapeDtypeStruct((NC, HIST), jnp.float32),
      mesh=_sc_mesh(),
      compiler_params=pltpu.CompilerParams(needs_layout_passes=False),
      scratch_types=[
          pltpu.VMEM((5008,), jnp.int32),
          pltpu.VMEM((HIST,), jnp.float32),
          pltpu.VMEM((640,), jnp.float32),
          pltpu.VMEM((640,), jnp.float32),
          pltpu.VMEM_SHARED((NS, HIST), jnp.float32),
      ],
  )
  def k(dst_hbm, out_hbm, dstb, hist, accs, tmp, banks):
    c = lax.axis_index("c")
    s = lax.axis_index("s")
    wid = c * NS + s

    pltpu.sync_copy(dst_hbm.at[pl.ds(wid * EPT, EPT)], dstb.at[pl.ds(0, EPT)])

    def zh(i, _):
      hist[pl.ds(i * 16, 16)] = jnp.zeros((16,), jnp.float32)
      return 0
    lax.fori_loop(0, HIST // 16, zh, 0)

    def body(i, _):
      base = i * 16
      valid = (base + _i16()) < EPT
      v = jnp.where(valid, dstb[pl.ds(base, 16)], 0)
      ks, _vs, om = plsc.sort_key_val(v, _i16(), mask=valid)
      ksan = jnp.where(om, ks, -1 - _i16())
      prev = _dyngather(ksan, jnp.maximum(_i16() - 1, 0))
      first = (_i16() == 0) | (ksan != prev)
      pos_first = jnp.where(first, _i16(), 16)
      sm = -lax.rev(plsc.cummax(lax.rev(-pos_first, (0,))), (0,))
      nxt = _dyngather(sm, jnp.minimum(_i16() + 1, 15))
      nxt = jnp.where(_i16() == 15, 16, nxt)
      runlen = (nxt - pos_first).astype(jnp.float32)
      plsc.addupdate_scatter(hist, [ks], runlen, mask=first & om)
      return 0
    lax.fori_loop(0, 313, body, 0)

    pltpu.sync_copy(hist, banks.at[s])
    plsc.subcore_barrier()

    pltpu.sync_copy(banks.at[0, pl.ds(s * 640, 640)], accs)
    def mb(b, _):
      pltpu.sync_copy(banks.at[b, pl.ds(s * 640, 640)], tmp)
      def av(i, _2):
        plsc.addupdate(accs.at[pl.ds(i * 16, 16)], tmp[pl.ds(i * 16, 16)])
        return 0
      lax.fori_loop(0, 40, av, 0)
      return 0
    lax.fori_loop(1, NS, mb, 0)
    pltpu.sync_copy(accs, out_hbm.at[c, pl.ds(s * 640, 640)])

  return k(edge_dst)


def _tc_y(x, Wg, deg_col):
  nb = pl.cdiv(N, 128)

  def body(x_ref, d_ref, wg_ref, y_ref, di_ref):
    di = lax.rsqrt(d_ref[...])
    y_ref[...] = jnp.dot(x_ref[...] * di, wg_ref[...],
                         preferred_element_type=jnp.float32)
    di_ref[...] = di

  return pl.pallas_call(
      body,
      grid=(nb,),
      in_specs=[
          pl.BlockSpec((128, D), lambda i: (i, 0)),
          pl.BlockSpec((128, 1), lambda i: (i, 0)),
          pl.BlockSpec((D, D), lambda i: (0, 0)),
      ],
      out_specs=[
          pl.BlockSpec((128, D), lambda i: (i, 0)),
          pl.BlockSpec((128, 1), lambda i: (i, 0)),
      ],
      out_shape=[
          jax.ShapeDtypeStruct((N, D), jnp.float32),
          jax.ShapeDtypeStruct((N, 1), jnp.float32),
      ],
  )(x, deg_col, Wg)


def _sc_agg(y, edge_src, edge_dst, node_index):
  out_type = (
      jax.ShapeDtypeStruct((S, D), jnp.float32),
      jax.ShapeDtypeStruct((S, D), jnp.float32),
      jax.ShapeDtypeStruct((S,), jnp.int32),
  )

  @functools.partial(
      pl.kernel,
      out_type=out_type,
      mesh=_sc_mesh(),
      compiler_params=pltpu.CompilerParams(needs_layout_passes=False),
      scratch_types=[
          pltpu.VMEM((N,), jnp.int32),
          pltpu.VMEM((S,), jnp.int32),
          pltpu.VMEM((5008,), jnp.int32),
          pltpu.VMEM((5008,), jnp.int32),
          pltpu.VMEM((SEL_CAP,), jnp.int32),
          pltpu.VMEM((SEL_CAP,), jnp.int32),
          pltpu.VMEM((SEL_CAP,), jnp.int32),
          pltpu.VMEM((SEL_CAP,), jnp.int32),
          pltpu.VMEM((OCAP,), jnp.int32),
          pltpu.VMEM((OCAP,), jnp.int32),
          pltpu.VMEM((128,), jnp.int32),
          pltpu.VMEM((128, D), jnp.float32),
          pltpu.VMEM((SPT + 1, D), jnp.float32),
          pltpu.VMEM((S,), jnp.int32),
          pltpu.VMEM((16,), jnp.int32),
          pltpu.VMEM((NS * 16,), jnp.int32),
          pltpu.VMEM_SHARED((NS * SEL_CAP,), jnp.int32),
          pltpu.VMEM_SHARED((NS * SEL_CAP,), jnp.int32),
          pltpu.VMEM_SHARED((NS * 16,), jnp.int32),
          pltpu.SemaphoreType.DMA,
      ],
  )
  def k(y_hbm, src_hbm, dst_hbm, ni_hbm, acc0_hbm, acc1_hbm, csel_hbm,
        slot, nbuf, srcb, dstb, selsrc, selslot, scs, scl, osrc, oslot,
        idxg, rows, acc, csb, cbuf, cntv, psrc, pslot, cbank, sem):
    c = lax.axis_index("c")
    s = lax.axis_index("s")
    wid = c * NS + s

    pltpu.sync_copy(ni_hbm, nbuf)
    pltpu.sync_copy(src_hbm.at[pl.ds(wid * EPT, EPT)],
                    srcb.at[pl.ds(0, EPT)])
    pltpu.sync_copy(dst_hbm.at[pl.ds(wid * EPT, EPT)],
                    dstb.at[pl.ds(0, EPT)])

    def zr(r, _):
      def zv(v, _2):
        acc[r, pl.ds(v * 16, 16)] = jnp.zeros((16,), jnp.float32)
        return 0
      lax.fori_loop(0, D // 16, zv, 0)
      return 0
    lax.fori_loop(0, SPT + 1, zr, 0)

    def si(i, _):
      slot[pl.ds(i * 16, 16)] = jnp.full((16,), -1, jnp.int32)
      return 0
    lax.fori_loop(0, N // 16, si, 0)
    def sw(i, _):
      nv = nbuf[pl.ds(i * 16, 16)]
      jv = jnp.full((16,), i * 16, jnp.int32) + _i16()
      plsc.store_scatter(slot, [nv], jv)
      return 0
    lax.fori_loop(0, S // 16, sw, 0)

    j0 = wid * (S // NW)
    for k2 in range(2):
      sv = nbuf[pl.ds(j0 + k2 * 16, 16)]
      selsrc[pl.ds(k2 * 16, 16)] = sv
      selslot[pl.ds(k2 * 16, 16)] = (
          jnp.full((16,), j0 + k2 * 16, jnp.int32) + _i16())

    def flt(i, cnt):
      base = i * 16
      valid = (base + _i16()) < EPT
      dstv = jnp.where(valid, dstb[pl.ds(base, 16)], 0)
      srcv = srcb[pl.ds(base, 16)]
      slotv = plsc.load_gather(slot, [dstv])
      m = valid & (slotv >= 0)
      plsc.store_compressed(selsrc.at[pl.ds(cnt, 16)], srcv, mask=m)
      plsc.store_compressed(selslot.at[pl.ds(cnt, 16)], slotv, mask=m)
      return cnt + jnp.max(plsc.all_reduce_population_count(m))
    cnt = lax.fori_loop(0, 313, flt, jnp.int32(32))

    cnt_al = (cnt // 16) * 16
    keep = _i16() < (cnt - cnt_al)
    v = selsrc[pl.ds(cnt_al, 16)]
    selsrc[pl.ds(cnt_al, 16)] = jnp.where(keep, v, 0)
    v = selslot[pl.ds(cnt_al, 16)]
    selslot[pl.ds(cnt_al, 16)] = jnp.where(keep, v, DUMP)
    def pad(k2, _):
      off = cnt_al + 16 + k2 * 16
      selsrc[pl.ds(off, 16)] = jnp.zeros((16,), jnp.int32)
      selslot[pl.ds(off, 16)] = jnp.full((16,), DUMP, jnp.int32)
      return 0
    lax.fori_loop(0, 8, pad, 0)
    cntp = ((cnt + 127) // 128) * 128

    def pub(g, _):
      go = pl.multiple_of(g * 1024, 8)
      pltpu.sync_copy(selsrc.at[pl.ds(go, 1024)],
                      psrc.at[pl.ds(s * SEL_CAP + go, 1024)])
      pltpu.sync_copy(selslot.at[pl.ds(go, 1024)],
                      pslot.at[pl.ds(s * SEL_CAP + go, 1024)])
      return 0
    lax.fori_loop(0, (cntp + 1023) // 1024, pub, 0)
    cbuf[pl.ds(0, 16)] = jnp.full((16,), cntp, jnp.int32)
    pltpu.sync_copy(cbuf, cbank.at[pl.ds(s * 16, 16)])
    plsc.subcore_barrier()
    pltpu.sync_copy(cbank, cntv)

    def flush(fp):
      fp = pl.multiple_of(fp, 16)
      def cp(k2, _):
        idxg[pl.ds(k2 * 16, 16)] = osrc[pl.ds(fp + k2 * 16, 16)]
        return 0
      lax.fori_loop(0, 8, cp, 0)
      pltpu.async_copy(y_hbm.at[idxg], rows, sem).wait()
      def pe(eb, _):
        sls = oslot[pl.ds(fp + eb * 16, 16)]
        for l in range(16):
          sle = sls[l]
          for vv in range(D // 16):
            plsc.addupdate(acc.at[sle, pl.ds(vv * 16, 16)],
                           rows[eb * 16 + l, pl.ds(vv * 16, 16)])
        return 0
      lax.fori_loop(0, 8, pe, 0)
      return 0

    def bank(b, carry):
      fp0, co0 = carry
      nb16 = cntv[pl.ds(b * 16, 16)]
      iters = nb16[0] // 16
      def cpy(g, _):
        go = pl.multiple_of(g * 1024, 8)
        pltpu.sync_copy(psrc.at[pl.ds(b * SEL_CAP + go, 1024)],
                        scs.at[pl.ds(go, 1024)])
        pltpu.sync_copy(pslot.at[pl.ds(b * SEL_CAP + go, 1024)],
                        scl.at[pl.ds(go, 1024)])
        return 0
      lax.fori_loop(0, (iters * 16 + 1023) // 1024, cpy, 0)
      def scan(i, car):
        fp, co = car
        slv = scl[pl.ds(i * 16, 16)]
        srv = scs[pl.ds(i * 16, 16)]
        m = (slv >> 6) == s
        plsc.store_compressed(osrc.at[pl.ds(co, 16)], srv, mask=m)
        plsc.store_compressed(oslot.at[pl.ds(co, 16)], slv & 63, mask=m)
        co = co + jnp.max(plsc.all_reduce_population_count(m))
        def do_flush(op):
          fp2, co2 = op
          flush(fp2)
          fp3 = fp2 + 128
          def compact(op2):
            fp4, co4 = op2
            fp5 = pl.multiple_of(fp4, 16)
            def cpc(k2, _):
              q = osrc[pl.ds(fp5 + k2 * 16, 16)]
              osrc[pl.ds(k2 * 16, 16)] = q
              q = oslot[pl.ds(fp5 + k2 * 16, 16)]
              oslot[pl.ds(k2 * 16, 16)] = q
              return 0
            lax.fori_loop(0, 9, cpc, 0)
            return (jnp.int32(0), co4 - fp4)
          return lax.cond(fp3 >= 1024, compact, lambda op2: op2, (fp3, co2))
        return lax.cond(co - fp >= 128, do_flush, lambda op: op, (fp, co))
      return lax.fori_loop(0, iters, scan, (fp0, co0))
    fp, co = lax.fori_loop(0, NS, bank, (jnp.int32(0), jnp.int32(0)))

    co_al = (co // 16) * 16
    keep2 = _i16() < (co - co_al)
    v2 = osrc[pl.ds(co_al, 16)]
    osrc[pl.ds(co_al, 16)] = jnp.where(keep2, v2, 0)
    v2 = oslot[pl.ds(co_al, 16)]
    oslot[pl.ds(co_al, 16)] = jnp.where(keep2, v2, SPT)
    def pad2(k2, _):
      off = co_al + 16 + k2 * 16
      osrc[pl.ds(off, 16)] = jnp.zeros((16,), jnp.int32)
      oslot[pl.ds(off, 16)] = jnp.full((16,), SPT, jnp.int32)
      return 0
    lax.fori_loop(0, 8, pad2, 0)
    cop = ((co + 127) // 128) * 128
    def dr(g, _):
      flush(fp + g * 128)
      return 0
    lax.fori_loop(0, (cop - fp) // 128, dr, 0)

    @pl.when(c == 0)
    def _():
      pltpu.sync_copy(acc.at[pl.ds(0, SPT)], acc0_hbm.at[pl.ds(s * SPT, SPT)])

    @pl.when(c == 1)
    def _():
      pltpu.sync_copy(acc.at[pl.ds(0, SPT)], acc1_hbm.at[pl.ds(s * SPT, SPT)])

    @pl.when(wid == 0)
    def _():
      def cw(i, _):
        nv = nbuf[pl.ds(i * 16, 16)]
        csb[pl.ds(i * 16, 16)] = plsc.load_gather(slot, [nv])
        return 0
      lax.fori_loop(0, S // 16, cw, 0)
      pltpu.sync_copy(csb, csel_hbm)

  return k(y, edge_src, edge_dst, node_index)


def _sc_sites(acc0, acc1, csel, node_index, dinv, bg):
  SPW = S // NW

  @functools.partial(
      pl.kernel,
      out_type=jax.ShapeDtypeStruct((S, D), jnp.float32),
      mesh=_sc_mesh(),
      compiler_params=pltpu.CompilerParams(needs_layout_passes=False),
      scratch_types=[
          pltpu.VMEM((SPW,), jnp.int32),
          pltpu.VMEM((SPW,), jnp.int32),
          pltpu.VMEM((N,), jnp.float32),
          pltpu.VMEM((SPW, D), jnp.float32),
          pltpu.VMEM((SPW, D), jnp.float32),
          pltpu.VMEM((SPW, D), jnp.float32),
          pltpu.VMEM((D,), jnp.float32),
          pltpu.SemaphoreType.DMA,
      ],
  )
  def k(a0_hbm, a1_hbm, cs_hbm, ni_hbm, di_hbm, bg_hbm, out_hbm,
        cb, nb, db, ra, rb, ob, bgb, sem):
    c = lax.axis_index("c")
    s = lax.axis_index("s")
    wid = c * NS + s
    j0 = wid * SPW

    pltpu.sync_copy(cs_hbm.at[pl.ds(j0, SPW)], cb)
    pltpu.sync_copy(ni_hbm.at[pl.ds(j0, SPW)], nb)
    pltpu.sync_copy(bg_hbm, bgb)
    pltpu.sync_copy(di_hbm, db)
    pltpu.async_copy(a0_hbm.at[cb], ra, sem).wait()
    pltpu.async_copy(a1_hbm.at[cb], rb, sem).wait()

    def site(i, _):
      nv = nb[pl.ds((i // 16) * 16, 16)]
      dv = plsc.load_gather(db, [nv])
      dsp = _dyngather(dv, jnp.full((16,), i % 16, jnp.int32))
      def vec(v, _2):
        a = ra[i, pl.ds(v * 16, 16)]
        b = rb[i, pl.ds(v * 16, 16)]
        g = bgb[pl.ds(v * 16, 16)]
        val = (a + b) * dsp + g
        ob[i, pl.ds(v * 16, 16)] = jnp.maximum(val, val * 0.01)
        return 0
      lax.fori_loop(0, D // 16, vec, 0)
      return 0
    lax.fori_loop(0, SPW, site, 0)

    pltpu.sync_copy(ob, out_hbm.at[pl.ds(j0, SPW)])

  return k(acc0, acc1, csel, node_index, dinv, bg)


def _tc_mlp(hs, W1, b1, W2, b2, W3, b3, Wo_pad, bo_pad):
  def body(h_ref, w1, b1r, w2, b2r, w3, b3r, wo, bor, o_ref):
    h = h_ref[...]
    h = jnp.dot(h, w1[...], preferred_element_type=jnp.float32) + b1r[...]
    h = jnp.maximum(h, h * 0.01)
    h = jnp.dot(h, w2[...], preferred_element_type=jnp.float32) + b2r[...]
    h = jnp.maximum(h, h * 0.01)
    h = jnp.dot(h, w3[...], preferred_element_type=jnp.float32) + b3r[...]
    h = jnp.maximum(h, h * 0.01)
    z = jnp.dot(h, wo[...], preferred_element_type=jnp.float32) + bor[...]
    col = lax.broadcasted_iota(jnp.int32, (128, 128), 1)
    z = jnp.where(col < 10, z, -1e30)
    z = z - jnp.max(z, axis=1, keepdims=True)
    e = jnp.exp(z)
    o_ref[...] = e / jnp.sum(e, axis=1, keepdims=True)

  full = lambda shape: pl.BlockSpec(shape, lambda i: tuple(0 for _ in shape))
  return pl.pallas_call(
      body,
      grid=(S // 128,),
      in_specs=[
          pl.BlockSpec((128, D), lambda i: (i, 0)),
          full((D, 128)), full((1, 128)),
          full((128, 128)), full((1, 128)),
          full((128, 64)), full((1, 64)),
          full((64, 128)), full((1, 128)),
      ],
      out_specs=pl.BlockSpec((128, 128), lambda i: (i, 0)),
      out_shape=jax.ShapeDtypeStruct((S, 128), jnp.float32),
  )(hs, W1, b1.reshape(1, -1), W2, b2.reshape(1, -1),
    W3, b3.reshape(1, -1), Wo_pad, bo_pad)


def kernel(x, edge_index, node_index, Wg, bg, W1, b1, W2, b2, W3, b3, Wo, bo):
  edge_src = edge_index[0]
  edge_dst = edge_index[1]
  deg_parts = _sc_deg(edge_dst)
  deg_col = (deg_parts[0, :N] + deg_parts[1, :N] + 1.0).reshape(N, 1)
  y, dinv_col = _tc_y(x, Wg, deg_col)
  acc0, acc1, csel = _sc_agg(y, edge_src, edge_dst, node_index)
  hs = _sc_sites(acc0, acc1, csel, node_index, dinv_col.reshape(N), bg)
  Wo_pad = jnp.pad(Wo, ((0, 0), (0, 128 - Wo.shape[1])))
  bo_pad = jnp.pad(bo, (0, 128 - bo.shape[0])).reshape(1, -1)
  out = _tc_mlp(hs, W1, b1, W2, b2, W3, b3, Wo_pad, bo_pad)
  return out[:, :bo.shape[0]]


# speedup vs baseline: 1.3921x; 1.0024x over previous
import functools

import jax
import jax.numpy as jnp
from jax import lax
from jax.experimental import pallas as pl
from jax.experimental.pallas import tpu as pltpu
from jax.experimental.pallas import tpu_sc as plsc

N = 10000
E = 160000
D = 256
S = 1024
NC = 2
NS = 16
NW = NC * NS
EPT = E // NW
HIST = 10240
SEL_CAP = 6144
DUMP = S
BIN_CAP = 5504
OCAP = 1328
SPT = S // NS

_sc_mesh = functools.partial(
    plsc.VectorSubcoreMesh, core_axis_name="c", subcore_axis_name="s",
    num_cores=NC, num_subcores=NS)


def _i16():
  return lax.iota(jnp.int32, 16)


def _dyngather(x, idx):
  return lax.gather(
      x, idx[:, None],
      dimension_numbers=lax.GatherDimensionNumbers(
          offset_dims=(), collapsed_slice_dims=(0,), start_index_map=(0,)),
      slice_sizes=(1,), mode=lax.GatherScatterMode.PROMISE_IN_BOUNDS)


def _sc_deg(edge_dst):
  @functools.partial(
      pl.kernel,
      out_type=jax.ShapeDtypeStruct((NC, HIST), jnp.float32),
      mesh=_sc_mesh(),
      compiler_params=pltpu.CompilerParams(needs_layout_passes=False),
      scratch_types=[
          pltpu.VMEM((5008,), jnp.int32),
          pltpu.VMEM((HIST,), jnp.float32),
          pltpu.VMEM((640,), jnp.float32),
          pltpu.VMEM((640,), jnp.float32),
          pltpu.VMEM_SHARED((NS, HIST), jnp.float32),
      ],
  )
  def k(dst_hbm, out_hbm, dstb, hist, accs, tmp, banks):
    c = lax.axis_index("c")
    s = lax.axis_index("s")
    wid = c * NS + s

    pltpu.sync_copy(dst_hbm.at[pl.ds(wid * EPT, EPT)], dstb.at[pl.ds(0, EPT)])

    def zh(i, _):
      hist[pl.ds(i * 16, 16)] = jnp.zeros((16,), jnp.float32)
      return 0
    lax.fori_loop(0, HIST // 16, zh, 0)

    def body(i, _):
      base = i * 16
      valid = (base + _i16()) < EPT
      v = jnp.where(valid, dstb[pl.ds(base, 16)], 0)
      ks, _vs, om = plsc.sort_key_val(v, _i16(), mask=valid)
      ksan = jnp.where(om, ks, -1 - _i16())
      prev = _dyngather(ksan, jnp.maximum(_i16() - 1, 0))
      first = (_i16() == 0) | (ksan != prev)
      pos_first = jnp.where(first, _i16(), 16)
      sm = -lax.rev(plsc.cummax(lax.rev(-pos_first, (0,))), (0,))
      nxt = _dyngather(sm, jnp.minimum(_i16() + 1, 15))
      nxt = jnp.where(_i16() == 15, 16, nxt)
      runlen = (nxt - pos_first).astype(jnp.float32)
      plsc.addupdate_scatter(hist, [ks], runlen, mask=first & om)
      return 0
    lax.fori_loop(0, 313, body, 0)

    pltpu.sync_copy(hist, banks.at[s])
    plsc.subcore_barrier()

    pltpu.sync_copy(banks.at[0, pl.ds(s * 640, 640)], accs)
    def mb(b, _):
      pltpu.sync_copy(banks.at[b, pl.ds(s * 640, 640)], tmp)
      def av(i, _2):
        plsc.addupdate(accs.at[pl.ds(i * 16, 16)], tmp[pl.ds(i * 16, 16)])
        return 0
      lax.fori_loop(0, 40, av, 0)
      return 0
    lax.fori_loop(1, NS, mb, 0)
    pltpu.sync_copy(accs, out_hbm.at[c, pl.ds(s * 640, 640)])

  return k(edge_dst)


def _tc_y(x, Wg, deg_col):
  nb = pl.cdiv(N, 128)

  def body(x_ref, d_ref, wg_ref, y_ref, di_ref):
    di = lax.rsqrt(d_ref[...])
    y_ref[...] = jnp.dot(x_ref[...] * di, wg_ref[...],
                         preferred_element_type=jnp.float32)
    di_ref[...] = di

  return pl.pallas_call(
      body,
      grid=(nb,),
      in_specs=[
          pl.BlockSpec((128, D), lambda i: (i, 0)),
          pl.BlockSpec((128, 1), lambda i: (i, 0)),
          pl.BlockSpec((D, D), lambda i: (0, 0)),
      ],
      out_specs=[
          pl.BlockSpec((128, D), lambda i: (i, 0)),
          pl.BlockSpec((128, 1), lambda i: (i, 0)),
      ],
      out_shape=[
          jax.ShapeDtypeStruct((N, D), jnp.float32),
          jax.ShapeDtypeStruct((N, 1), jnp.float32),
      ],
  )(x, deg_col, Wg)


def _sc_agg(y, edge_src, edge_dst, node_index, dinv):
  out_type = (
      jax.ShapeDtypeStruct((S, D), jnp.float32),
      jax.ShapeDtypeStruct((S, D), jnp.float32),
      jax.ShapeDtypeStruct((S,), jnp.int32),
      jax.ShapeDtypeStruct((S,), jnp.float32),
  )

  @functools.partial(
      pl.kernel,
      out_type=out_type,
      mesh=_sc_mesh(),
      compiler_params=pltpu.CompilerParams(needs_layout_passes=False),
      scratch_types=[
          pltpu.VMEM((N,), jnp.int32),
          pltpu.VMEM((S,), jnp.int32),
          pltpu.VMEM((5008,), jnp.int32),
          pltpu.VMEM((5008,), jnp.int32),
          pltpu.VMEM((SEL_CAP,), jnp.int32),
          pltpu.VMEM((SEL_CAP,), jnp.int32),
          pltpu.VMEM((SEL_CAP,), jnp.int32),
          pltpu.VMEM((SEL_CAP,), jnp.int32),
          pltpu.VMEM((OCAP,), jnp.int32),
          pltpu.VMEM((OCAP,), jnp.int32),
          pltpu.VMEM((128,), jnp.int32),
          pltpu.VMEM((128, D), jnp.float32),
          pltpu.VMEM((SPT + 1, D), jnp.float32),
          pltpu.VMEM((S,), jnp.int32),
          pltpu.VMEM((N,), jnp.float32),
          pltpu.VMEM((S,), jnp.float32),
          pltpu.VMEM((16,), jnp.int32),
          pltpu.VMEM((NS * 16,), jnp.int32),
          pltpu.VMEM_SHARED((NS * SEL_CAP,), jnp.int32),
          pltpu.VMEM_SHARED((NS * SEL_CAP,), jnp.int32),
          pltpu.VMEM_SHARED((NS * 16,), jnp.int32),
          pltpu.SemaphoreType.DMA,
      ],
  )
  def k(y_hbm, src_hbm, dst_hbm, ni_hbm, di_hbm, acc0_hbm, acc1_hbm,
        csel_hbm, dis_hbm,
        slot, nbuf, srcb, dstb, selsrc, selslot, scs, scl, osrc, oslot,
        idxg, rows, acc, csb, dbuf, dsb, cbuf, cntv, psrc, pslot, cbank,
        sem):
    c = lax.axis_index("c")
    s = lax.axis_index("s")
    wid = c * NS + s

    pltpu.sync_copy(ni_hbm, nbuf)
    pltpu.sync_copy(src_hbm.at[pl.ds(wid * EPT, EPT)],
                    srcb.at[pl.ds(0, EPT)])
    pltpu.sync_copy(dst_hbm.at[pl.ds(wid * EPT, EPT)],
                    dstb.at[pl.ds(0, EPT)])

    def zr(r, _):
      def zv(v, _2):
        acc[r, pl.ds(v * 16, 16)] = jnp.zeros((16,), jnp.float32)
        return 0
      lax.fori_loop(0, D // 16, zv, 0)
      return 0
    lax.fori_loop(0, SPT + 1, zr, 0)

    def si(i, _):
      slot[pl.ds(i * 16, 16)] = jnp.full((16,), -1, jnp.int32)
      return 0
    lax.fori_loop(0, N // 16, si, 0)
    def sw(i, _):
      nv = nbuf[pl.ds(i * 16, 16)]
      jv = jnp.full((16,), i * 16, jnp.int32) + _i16()
      plsc.store_scatter(slot, [nv], jv)
      return 0
    lax.fori_loop(0, S // 16, sw, 0)

    j0 = wid * (S // NW)
    for k2 in range(2):
      sv = nbuf[pl.ds(j0 + k2 * 16, 16)]
      selsrc[pl.ds(k2 * 16, 16)] = sv
      selslot[pl.ds(k2 * 16, 16)] = (
          jnp.full((16,), j0 + k2 * 16, jnp.int32) + _i16())

    def flt(i, cnt):
      base = i * 16
      valid = (base + _i16()) < EPT
      dstv = jnp.where(valid, dstb[pl.ds(base, 16)], 0)
      srcv = srcb[pl.ds(base, 16)]
      slotv = plsc.load_gather(slot, [dstv])
      m = valid & (slotv >= 0)
      plsc.store_compressed(selsrc.at[pl.ds(cnt, 16)], srcv, mask=m)
      plsc.store_compressed(selslot.at[pl.ds(cnt, 16)], slotv, mask=m)
      return cnt + jnp.max(plsc.all_reduce_population_count(m))
    cnt = lax.fori_loop(0, 313, flt, jnp.int32(32))

    cnt_al = (cnt // 16) * 16
    keep = _i16() < (cnt - cnt_al)
    v = selsrc[pl.ds(cnt_al, 16)]
    selsrc[pl.ds(cnt_al, 16)] = jnp.where(keep, v, 0)
    v = selslot[pl.ds(cnt_al, 16)]
    selslot[pl.ds(cnt_al, 16)] = jnp.where(keep, v, DUMP)
    def pad(k2, _):
      off = cnt_al + 16 + k2 * 16
      selsrc[pl.ds(off, 16)] = jnp.zeros((16,), jnp.int32)
      selslot[pl.ds(off, 16)] = jnp.full((16,), DUMP, jnp.int32)
      return 0
    lax.fori_loop(0, 8, pad, 0)
    cntp = ((cnt + 127) // 128) * 128

    def pub(g, _):
      go = pl.multiple_of(g * 1024, 8)
      pltpu.sync_copy(selsrc.at[pl.ds(go, 1024)],
                      psrc.at[pl.ds(s * SEL_CAP + go, 1024)])
      pltpu.sync_copy(selslot.at[pl.ds(go, 1024)],
                      pslot.at[pl.ds(s * SEL_CAP + go, 1024)])
      return 0
    lax.fori_loop(0, (cntp + 1023) // 1024, pub, 0)
    cbuf[pl.ds(0, 16)] = jnp.full((16,), cntp, jnp.int32)
    pltpu.sync_copy(cbuf, cbank.at[pl.ds(s * 16, 16)])
    plsc.subcore_barrier()
    pltpu.sync_copy(cbank, cntv)

    def flush(fp):
      fp = pl.multiple_of(fp, 16)
      def cp(k2, _):
        idxg[pl.ds(k2 * 16, 16)] = osrc[pl.ds(fp + k2 * 16, 16)]
        return 0
      lax.fori_loop(0, 8, cp, 0)
      pltpu.async_copy(y_hbm.at[idxg], rows, sem).wait()
      def pe(eb, _):
        sls = oslot[pl.ds(fp + eb * 16, 16)]
        for l in range(16):
          sle = sls[l]
          for vv in range(D // 16):
            plsc.addupdate(acc.at[sle, pl.ds(vv * 16, 16)],
                           rows[eb * 16 + l, pl.ds(vv * 16, 16)])
        return 0
      lax.fori_loop(0, 8, pe, 0)
      return 0

    def bank(b, carry):
      fp0, co0 = carry
      nb16 = cntv[pl.ds(b * 16, 16)]
      iters = nb16[0] // 16
      def cpy(g, _):
        go = pl.multiple_of(g * 1024, 8)
        pltpu.sync_copy(psrc.at[pl.ds(b * SEL_CAP + go, 1024)],
                        scs.at[pl.ds(go, 1024)])
        pltpu.sync_copy(pslot.at[pl.ds(b * SEL_CAP + go, 1024)],
                        scl.at[pl.ds(go, 1024)])
        return 0
      lax.fori_loop(0, (iters * 16 + 1023) // 1024, cpy, 0)
      def scan(i, car):
        fp, co = car
        slv = scl[pl.ds(i * 16, 16)]
        srv = scs[pl.ds(i * 16, 16)]
        m = (slv >> 6) == s
        plsc.store_compressed(osrc.at[pl.ds(co, 16)], srv, mask=m)
        plsc.store_compressed(oslot.at[pl.ds(co, 16)], slv & 63, mask=m)
        co = co + jnp.max(plsc.all_reduce_population_count(m))
        def do_flush(op):
          fp2, co2 = op
          flush(fp2)
          fp3 = fp2 + 128
          def compact(op2):
            fp4, co4 = op2
            fp5 = pl.multiple_of(fp4, 16)
            def cpc(k2, _):
              q = osrc[pl.ds(fp5 + k2 * 16, 16)]
              osrc[pl.ds(k2 * 16, 16)] = q
              q = oslot[pl.ds(fp5 + k2 * 16, 16)]
              oslot[pl.ds(k2 * 16, 16)] = q
              return 0
            lax.fori_loop(0, 9, cpc, 0)
            return (jnp.int32(0), co4 - fp4)
          return lax.cond(fp3 >= 1024, compact, lambda op2: op2, (fp3, co2))
        return lax.cond(co - fp >= 128, do_flush, lambda op: op, (fp, co))
      return lax.fori_loop(0, iters, scan, (fp0, co0))
    fp, co = lax.fori_loop(0, NS, bank, (jnp.int32(0), jnp.int32(0)))

    co_al = (co // 16) * 16
    keep2 = _i16() < (co - co_al)
    v2 = osrc[pl.ds(co_al, 16)]
    osrc[pl.ds(co_al, 16)] = jnp.where(keep2, v2, 0)
    v2 = oslot[pl.ds(co_al, 16)]
    oslot[pl.ds(co_al, 16)] = jnp.where(keep2, v2, SPT)
    def pad2(k2, _):
      off = co_al + 16 + k2 * 16
      osrc[pl.ds(off, 16)] = jnp.zeros((16,), jnp.int32)
      oslot[pl.ds(off, 16)] = jnp.full((16,), SPT, jnp.int32)
      return 0
    lax.fori_loop(0, 8, pad2, 0)
    cop = ((co + 127) // 128) * 128
    def dr(g, _):
      flush(fp + g * 128)
      return 0
    lax.fori_loop(0, (cop - fp) // 128, dr, 0)

    @pl.when(c == 0)
    def _():
      pltpu.sync_copy(acc.at[pl.ds(0, SPT)], acc0_hbm.at[pl.ds(s * SPT, SPT)])

    @pl.when(c == 1)
    def _():
      pltpu.sync_copy(acc.at[pl.ds(0, SPT)], acc1_hbm.at[pl.ds(s * SPT, SPT)])

    @pl.when(wid == 0)
    def _():
      def cw(i, _):
        nv = nbuf[pl.ds(i * 16, 16)]
        csb[pl.ds(i * 16, 16)] = plsc.load_gather(slot, [nv])
        return 0
      lax.fori_loop(0, S // 16, cw, 0)
      pltpu.sync_copy(csb, csel_hbm)

    @pl.when(wid == 1)
    def _():
      pltpu.sync_copy(di_hbm, dbuf)
      def dw(i, _):
        nv = nbuf[pl.ds(i * 16, 16)]
        dsb[pl.ds(i * 16, 16)] = plsc.load_gather(dbuf, [nv])
        return 0
      lax.fori_loop(0, S // 16, dw, 0)
      pltpu.sync_copy(dsb, dis_hbm)

  return k(y, edge_src, edge_dst, node_index, dinv)


def _tc_mlp(acc0, acc1, csel_col, dis_col, bg, W1, b1, W2, b2, W3, b3,
            Wo_pad, bo_pad):
  def body(a0_ref, a1_ref, cs_ref, di_ref, bg_ref,
           w1, b1r, w2, b2r, w3, b3r, wo, bor, o_ref):
    P = (lax.broadcasted_iota(jnp.int32, (128, S), 1)
         == cs_ref[...]).astype(jnp.float32)
    asum = a0_ref[...] + a1_ref[...]
    z = jnp.dot(P, asum, preferred_element_type=jnp.float32)
    h = z * di_ref[...] + bg_ref[...]
    h = jnp.maximum(h, h * 0.01)
    h = jnp.dot(h, w1[...], preferred_element_type=jnp.float32) + b1r[...]
    h = jnp.maximum(h, h * 0.01)
    h = jnp.dot(h, w2[...], preferred_element_type=jnp.float32) + b2r[...]
    h = jnp.maximum(h, h * 0.01)
    h = jnp.dot(h, w3[...], preferred_element_type=jnp.float32) + b3r[...]
    h = jnp.maximum(h, h * 0.01)
    z = jnp.dot(h, wo[...], preferred_element_type=jnp.float32) + bor[...]
    col = lax.broadcasted_iota(jnp.int32, (128, 128), 1)
    z = jnp.where(col < 10, z, -1e30)
    z = z - jnp.max(z, axis=1, keepdims=True)
    e = jnp.exp(z)
    o_ref[...] = e / jnp.sum(e, axis=1, keepdims=True)

  full = lambda shape: pl.BlockSpec(shape, lambda i: tuple(0 for _ in shape))
  return pl.pallas_call(
      body,
      grid=(S // 128,),
      in_specs=[
          full((S, D)), full((S, D)),
          pl.BlockSpec((128, 1), lambda i: (i, 0)),
          pl.BlockSpec((128, 1), lambda i: (i, 0)),
          full((1, D)),
          full((D, 128)), full((1, 128)),
          full((128, 128)), full((1, 128)),
          full((128, 64)), full((1, 64)),
          full((64, 128)), full((1, 128)),
      ],
      out_specs=pl.BlockSpec((128, 128), lambda i: (i, 0)),
      out_shape=jax.ShapeDtypeStruct((S, 128), jnp.float32),
  )(acc0, acc1, csel_col, dis_col, bg.reshape(1, -1),
    W1, b1.reshape(1, -1), W2, b2.reshape(1, -1),
    W3, b3.reshape(1, -1), Wo_pad, bo_pad)


def kernel(x, edge_index, node_index, Wg, bg, W1, b1, W2, b2, W3, b3, Wo, bo):
  edge_src = edge_index[0]
  edge_dst = edge_index[1]
  deg_parts = _sc_deg(edge_dst)
  deg_col = (deg_parts[0, :N] + deg_parts[1, :N] + 1.0).reshape(N, 1)
  y, dinv_col = _tc_y(x, Wg, deg_col)
  acc0, acc1, csel, dis = _sc_agg(y, edge_src, edge_dst, node_index,
                                  dinv_col.reshape(N))
  Wo_pad = jnp.pad(Wo, ((0, 0), (0, 128 - Wo.shape[1])))
  bo_pad = jnp.pad(bo, (0, 128 - bo.shape[0])).reshape(1, -1)
  out = _tc_mlp(acc0, acc1, csel.reshape(S, 1), dis.reshape(S, 1), bg,
                W1, b1, W2, b2, W3, b3, Wo_pad, bo_pad)
  return out[:, :bo.shape[0]]
